# baseline (speedup 1.0000x reference)
import functools

import jax
import jax.numpy as jnp
from jax import lax
from jax.experimental import pallas as pl
from jax.experimental.pallas import tpu as pltpu
from jax.experimental.pallas import tpu_sc as plsc

NUM_CORES = 2
NUM_SUBCORES = 16
LANES = 16
NUM_WORKERS = NUM_CORES * NUM_SUBCORES

BATCH = 16384
EMBED = 32
B_PER_W = BATCH // NUM_WORKERS


def _body(user_hbm, item_hbm, utab_hbm, itab_hbm, out_hbm,
          idx_u, idx_i, rows_u, rows_i, out_v, sem_u, sem_i):
  wid = lax.axis_index("s") * NUM_CORES + lax.axis_index("c")
  base = wid * B_PER_W

  pltpu.sync_copy(user_hbm.at[pl.ds(base, B_PER_W)], idx_u)
  pltpu.sync_copy(item_hbm.at[pl.ds(base, B_PER_W)], idx_i)

  cu = pltpu.async_copy(utab_hbm.at[idx_u], rows_u, sem_u)
  ci = pltpu.async_copy(itab_hbm.at[idx_i], rows_i, sem_i)
  cu.wait()
  ci.wait()

  iota = lax.iota(jnp.int32, LANES)

  def group(g, carry):
    rid = g * LANES + iota
    acc = jnp.zeros((LANES,), jnp.float32)
    for e in range(EMBED):
      col = jnp.full((LANES,), e, jnp.int32)
      u = plsc.load_gather(rows_u, [rid, col])
      v = plsc.load_gather(rows_i, [rid, col])
      acc = acc + u * v
    out_v[pl.ds(g * LANES, LANES)] = 1.0 / (1.0 + jnp.exp(-acc))
    return carry

  lax.fori_loop(0, B_PER_W // LANES, group, 0)

  pltpu.sync_copy(out_v, out_hbm.at[pl.ds(base, B_PER_W)])


@jax.jit
def kernel(user, item, user_table, item_table):
  mesh = plsc.VectorSubcoreMesh(core_axis_name="c", subcore_axis_name="s")
  return pl.kernel(
      _body,
      out_type=jax.ShapeDtypeStruct((BATCH,), jnp.float32),
      mesh=mesh,
      compiler_params=pltpu.CompilerParams(
          needs_layout_passes=False, use_tc_tiling_on_sc=False),
      scratch_types=[
          pltpu.VMEM((B_PER_W,), jnp.int32),
          pltpu.VMEM((B_PER_W,), jnp.int32),
          pltpu.VMEM((B_PER_W, EMBED), jnp.float32),
          pltpu.VMEM((B_PER_W, EMBED), jnp.float32),
          pltpu.VMEM((B_PER_W,), jnp.float32),
          pltpu.SemaphoreType.DMA,
          pltpu.SemaphoreType.DMA,
      ],
  )(user, item, user_table, item_table)


# speedup vs baseline: 4.0114x; 4.0114x over previous
import functools

import jax
import jax.numpy as jnp
from jax import lax
from jax.experimental import pallas as pl
from jax.experimental.pallas import tpu as pltpu
from jax.experimental.pallas import tpu_sc as plsc

NUM_CORES = 2
NUM_SUBCORES = 16
LANES = 16
NUM_WORKERS = NUM_CORES * NUM_SUBCORES

BATCH = 16384
EMBED = 32
B_PER_W = BATCH // NUM_WORKERS
NBUF = 8
STAGE = 4
N_STAGES = B_PER_W // STAGE
IDX_PAD = B_PER_W + LANES


def _body(user_hbm, item_hbm, utab_hbm, itab_hbm, out_hbm,
          idx_u, idx_i, ub, ib, out_v, *sems):
  sem_u = sems[:NBUF]
  sem_i = sems[NBUF:]
  wid = lax.axis_index("s") * NUM_CORES + lax.axis_index("c")
  base = wid * B_PER_W

  pltpu.sync_copy(user_hbm.at[pl.ds(base, B_PER_W)],
                  idx_u.at[pl.ds(0, B_PER_W)])
  pltpu.sync_copy(item_hbm.at[pl.ds(base, B_PER_W)],
                  idx_i.at[pl.ds(0, B_PER_W)])

  iota = lax.iota(jnp.int32, LANES)

  def col_base(idx_vec, j):
    r = idx_vec[j]
    c = pl.multiple_of(r // 128 * 128, 128)
    return c, r - c

  def issue_stage(stage, slots):
    ru = idx_u[pl.ds(stage * STAGE, LANES)]
    ri = idx_i[pl.ds(stage * STAGE, LANES)]
    for j in range(STAGE):
      s = slots[j]
      cu, _ = col_base(ru, j)
      ci, _ = col_base(ri, j)
      pltpu.async_copy(utab_hbm.at[:, pl.ds(cu, 128)], ub.at[s], sem_u[s])
      pltpu.async_copy(itab_hbm.at[:, pl.ds(ci, 128)], ib.at[s], sem_i[s])

  def consume_stage(stage, slots, outacc):
    ru = idx_u[pl.ds(stage * STAGE, LANES)]
    ri = idx_i[pl.ds(stage * STAGE, LANES)]
    dummy = utab_hbm.at[:, pl.ds(0, 128)]
    for j in range(STAGE):
      s = slots[j]
      pltpu.make_async_copy(dummy, ub.at[s], sem_u[s]).wait()
      pltpu.make_async_copy(dummy, ib.at[s], sem_i[s]).wait()
      _, lu = col_base(ru, j)
      _, li = col_base(ri, j)
      svec = jnp.full((LANES,), s, jnp.int32)
      luv = jnp.full((LANES,), lu, jnp.int32)
      liv = jnp.full((LANES,), li, jnp.int32)
      u_lo = plsc.load_gather(ub, [svec, iota, luv])
      u_hi = plsc.load_gather(ub, [svec, iota + LANES, luv])
      i_lo = plsc.load_gather(ib, [svec, iota, liv])
      i_hi = plsc.load_gather(ib, [svec, iota + LANES, liv])
      dot = jnp.sum(u_lo * i_lo + u_hi * i_hi)
      n = stage * STAGE + j
      outacc = jnp.where(iota == n % LANES,
                         jnp.full((LANES,), dot, jnp.float32), outacc)
    return outacc

  lo = tuple(range(STAGE))
  hi = tuple(range(STAGE, NBUF))

  issue_stage(0, lo)

  def step(h2, outacc):
    issue_stage(2 * h2 + 1, hi)
    outacc = consume_stage(2 * h2, lo, outacc)

    @pl.when(h2 < N_STAGES // 2 - 1)
    def _():
      issue_stage(2 * h2 + 2, lo)

    outacc = consume_stage(2 * h2 + 1, hi, outacc)

    @pl.when(h2 % 2 == 1)
    def _():
      out_v[pl.ds(h2 // 2 * LANES, LANES)] = 1.0 / (1.0 + jnp.exp(-outacc))

    return jnp.where(h2 % 2 == 1, jnp.zeros((LANES,), jnp.float32), outacc)

  lax.fori_loop(0, N_STAGES // 2, step, jnp.zeros((LANES,), jnp.float32))

  pltpu.sync_copy(out_v, out_hbm.at[pl.ds(base, B_PER_W)])


@jax.jit
def kernel(user, item, user_table, item_table):
  mesh = plsc.VectorSubcoreMesh(core_axis_name="c", subcore_axis_name="s")
  return pl.kernel(
      _body,
      out_type=jax.ShapeDtypeStruct((BATCH,), jnp.float32),
      mesh=mesh,
      compiler_params=pltpu.CompilerParams(
          use_tc_tiling_on_sc=True, needs_layout_passes=False),
      scratch_types=[
          pltpu.VMEM((IDX_PAD,), jnp.int32),
          pltpu.VMEM((IDX_PAD,), jnp.int32),
          pltpu.VMEM((NBUF, EMBED, 128), jnp.float32),
          pltpu.VMEM((NBUF, EMBED, 128), jnp.float32),
          pltpu.VMEM((B_PER_W,), jnp.float32),
      ] + [pltpu.SemaphoreType.DMA] * (2 * NBUF),
  )(user, item, user_table.T, item_table.T)


# speedup vs baseline: 4.0231x; 1.0029x over previous
import functools

import jax
import jax.numpy as jnp
from jax import lax
from jax.experimental import pallas as pl
from jax.experimental.pallas import tpu as pltpu
from jax.experimental.pallas import tpu_sc as plsc

NUM_CORES = 2
NUM_SUBCORES = 16
LANES = 16
NUM_WORKERS = NUM_CORES * NUM_SUBCORES

BATCH = 16384
EMBED = 32
B_PER_W = BATCH // NUM_WORKERS
NBUF = 12
STAGE = 4
N_STAGES = B_PER_W // STAGE
IDX_PAD = B_PER_W + LANES


def _body(user_hbm, item_hbm, utab_hbm, itab_hbm, out_hbm,
          idx_u, idx_i, ub, ib, out_v, *sems):
  sem_u = sems[:NBUF]
  sem_i = sems[NBUF:]
  wid = lax.axis_index("s") * NUM_CORES + lax.axis_index("c")
  base = wid * B_PER_W

  pltpu.sync_copy(user_hbm.at[pl.ds(base, B_PER_W)],
                  idx_u.at[pl.ds(0, B_PER_W)])
  pltpu.sync_copy(item_hbm.at[pl.ds(base, B_PER_W)],
                  idx_i.at[pl.ds(0, B_PER_W)])

  iota = lax.iota(jnp.int32, LANES)

  def col_base(idx_vec, j):
    r = idx_vec[j]
    c = pl.multiple_of(r // 128 * 128, 128)
    return c, r - c

  def issue_stage(stage, slots):
    ru = idx_u[pl.ds(stage * STAGE, LANES)]
    ri = idx_i[pl.ds(stage * STAGE, LANES)]
    for j in range(STAGE):
      s = slots[j]
      cu, _ = col_base(ru, j)
      ci, _ = col_base(ri, j)
      pltpu.async_copy(utab_hbm.at[:, pl.ds(cu, 128)], ub.at[s], sem_u[s])
      pltpu.async_copy(itab_hbm.at[:, pl.ds(ci, 128)], ib.at[s], sem_i[s])

  def consume_stage(stage, slots, outacc):
    ru = idx_u[pl.ds(stage * STAGE, LANES)]
    ri = idx_i[pl.ds(stage * STAGE, LANES)]
    dummy = utab_hbm.at[:, pl.ds(0, 128)]
    for j in range(STAGE):
      s = slots[j]
      pltpu.make_async_copy(dummy, ub.at[s], sem_u[s]).wait()
      pltpu.make_async_copy(dummy, ib.at[s], sem_i[s]).wait()
      _, lu = col_base(ru, j)
      _, li = col_base(ri, j)
      svec = jnp.full((LANES,), s, jnp.int32)
      luv = jnp.full((LANES,), lu, jnp.int32)
      liv = jnp.full((LANES,), li, jnp.int32)
      u_lo = plsc.load_gather(ub, [svec, iota, luv])
      u_hi = plsc.load_gather(ub, [svec, iota + LANES, luv])
      i_lo = plsc.load_gather(ib, [svec, iota, liv])
      i_hi = plsc.load_gather(ib, [svec, iota + LANES, liv])
      dot = jnp.sum(u_lo * i_lo + u_hi * i_hi)
      n = stage * STAGE + j
      outacc = jnp.where(iota == n % LANES,
                         jnp.full((LANES,), dot, jnp.float32), outacc)
    return outacc

  bank_a = tuple(range(0, 4))
  bank_b = tuple(range(4, 8))
  bank_c = tuple(range(8, 12))

  def flush(stage, outacc):
    @pl.when(stage % 4 == 3)
    def _():
      out_v[pl.ds(stage // 4 * LANES, LANES)] = (
          1.0 / (1.0 + jnp.exp(-outacc)))

    return jnp.where(stage % 4 == 3,
                     jnp.zeros((LANES,), jnp.float32), outacc)

  issue_stage(0, bank_a)
  issue_stage(1, bank_b)

  def step(h3, outacc):
    s0 = 3 * h3
    issue_stage(s0 + 2, bank_c)
    outacc = flush(s0, consume_stage(s0, bank_a, outacc))
    issue_stage(s0 + 3, bank_a)
    outacc = flush(s0 + 1, consume_stage(s0 + 1, bank_b, outacc))
    issue_stage(s0 + 4, bank_b)
    outacc = flush(s0 + 2, consume_stage(s0 + 2, bank_c, outacc))
    return outacc

  outacc = lax.fori_loop(0, (N_STAGES - 2) // 3, step,
                         jnp.zeros((LANES,), jnp.float32))
  outacc = flush(jnp.int32(N_STAGES - 2),
                 consume_stage(N_STAGES - 2, bank_a, outacc))
  flush(jnp.int32(N_STAGES - 1),
        consume_stage(N_STAGES - 1, bank_b, outacc))

  pltpu.sync_copy(out_v, out_hbm.at[pl.ds(base, B_PER_W)])


@jax.jit
def kernel(user, item, user_table, item_table):
  mesh = plsc.VectorSubcoreMesh(core_axis_name="c", subcore_axis_name="s")
  return pl.kernel(
      _body,
      out_type=jax.ShapeDtypeStruct((BATCH,), jnp.float32),
      mesh=mesh,
      compiler_params=pltpu.CompilerParams(
          use_tc_tiling_on_sc=True, needs_layout_passes=False),
      scratch_types=[
          pltpu.VMEM((IDX_PAD,), jnp.int32),
          pltpu.VMEM((IDX_PAD,), jnp.int32),
          pltpu.VMEM((NBUF, EMBED, 128), jnp.float32),
          pltpu.VMEM((NBUF, EMBED, 128), jnp.float32),
          pltpu.VMEM((B_PER_W,), jnp.float32),
      ] + [pltpu.SemaphoreType.DMA] * (2 * NBUF),
  )(user, item, user_table.T, item_table.T)


# speedup vs baseline: 4.0281x; 1.0012x over previous
import functools

import jax
import jax.numpy as jnp
from jax import lax
from jax.experimental import pallas as pl
from jax.experimental.pallas import tpu as pltpu
from jax.experimental.pallas import tpu_sc as plsc

NUM_CORES = 2
NUM_SUBCORES = 16
LANES = 16
NUM_WORKERS = NUM_CORES * NUM_SUBCORES

BATCH = 16384
EMBED = 32
B_PER_W = BATCH // NUM_WORKERS
NBUF = 8
STAGE = 4
N_STAGES = B_PER_W // STAGE
IDX_PAD = B_PER_W + LANES


def _body(user_hbm, item_hbm, utab_hbm, itab_hbm, out_hbm,
          idx_u, idx_i, ub, ib, out_v, *sems):
  sem_u = sems[:NBUF]
  sem_i = sems[NBUF:]
  wid = lax.axis_index("s") * NUM_CORES + lax.axis_index("c")
  base = wid * B_PER_W

  pltpu.sync_copy(user_hbm.at[pl.ds(base, B_PER_W)],
                  idx_u.at[pl.ds(0, B_PER_W)])
  pltpu.sync_copy(item_hbm.at[pl.ds(base, B_PER_W)],
                  idx_i.at[pl.ds(0, B_PER_W)])

  iota = lax.iota(jnp.int32, LANES)

  def col_base(idx_vec, j):
    r = idx_vec[j]
    c = pl.multiple_of(r // 128 * 128, 128)
    return c, r - c

  def issue_stage(stage, slots):
    ru = idx_u[pl.ds(stage * STAGE, LANES)]
    ri = idx_i[pl.ds(stage * STAGE, LANES)]
    for j in range(STAGE):
      s = slots[j]
      cu, _ = col_base(ru, j)
      ci, _ = col_base(ri, j)
      pltpu.async_copy(utab_hbm.at[:, pl.ds(cu, 128)], ub.at[s], sem_u[s])
      pltpu.async_copy(itab_hbm.at[:, pl.ds(ci, 128)], ib.at[s], sem_i[s])

  def consume_stage(stage, slots, outacc):
    ru = idx_u[pl.ds(stage * STAGE, LANES)]
    ri = idx_i[pl.ds(stage * STAGE, LANES)]
    dummy = utab_hbm.at[:, pl.ds(0, 128)]
    for j in range(STAGE):
      s = slots[j]
      pltpu.make_async_copy(dummy, ub.at[s], sem_u[s]).wait()
      pltpu.make_async_copy(dummy, ib.at[s], sem_i[s]).wait()
      _, lu = col_base(ru, j)
      _, li = col_base(ri, j)
      svec = jnp.full((LANES,), s, jnp.int32)
      luv = jnp.full((LANES,), lu, jnp.int32)
      liv = jnp.full((LANES,), li, jnp.int32)
      u_lo = plsc.load_gather(ub, [svec, iota, luv])
      u_hi = plsc.load_gather(ub, [svec, iota + LANES, luv])
      i_lo = plsc.load_gather(ib, [svec, iota, liv])
      i_hi = plsc.load_gather(ib, [svec, iota + LANES, liv])
      dot = jnp.sum(u_lo * i_lo + u_hi * i_hi)
      n = stage * STAGE + j
      outacc = jnp.where(iota == n % LANES,
                         jnp.full((LANES,), dot, jnp.float32), outacc)
    return outacc

  lo = tuple(range(STAGE))
  hi = tuple(range(STAGE, NBUF))

  issue_stage(0, lo)

  def step(h2, outacc):
    issue_stage(2 * h2 + 1, hi)
    outacc = consume_stage(2 * h2, lo, outacc)

    @pl.when(h2 < N_STAGES // 2 - 1)
    def _():
      issue_stage(2 * h2 + 2, lo)

    outacc = consume_stage(2 * h2 + 1, hi, outacc)

    @pl.when(h2 % 2 == 1)
    def _():
      out_v[pl.ds(h2 // 2 * LANES, LANES)] = 1.0 / (1.0 + jnp.exp(-outacc))

    return jnp.where(h2 % 2 == 1, jnp.zeros((LANES,), jnp.float32), outacc)

  lax.fori_loop(0, N_STAGES // 2, step, jnp.zeros((LANES,), jnp.float32))

  pltpu.sync_copy(out_v, out_hbm.at[pl.ds(base, B_PER_W)])


@jax.jit
def kernel(user, item, user_table, item_table):
  mesh = plsc.VectorSubcoreMesh(core_axis_name="c", subcore_axis_name="s")
  return pl.kernel(
      _body,
      out_type=jax.ShapeDtypeStruct((BATCH,), jnp.float32),
      mesh=mesh,
      compiler_params=pltpu.CompilerParams(
          use_tc_tiling_on_sc=True, needs_layout_passes=False),
      scratch_types=[
          pltpu.VMEM((IDX_PAD,), jnp.int32),
          pltpu.VMEM((IDX_PAD,), jnp.int32),
          pltpu.VMEM((NBUF, EMBED, 128), jnp.float32),
          pltpu.VMEM((NBUF, EMBED, 128), jnp.float32),
          pltpu.VMEM((B_PER_W,), jnp.float32),
      ] + [pltpu.SemaphoreType.DMA] * (2 * NBUF),
  )(user, item, user_table.T, item_table.T)
